# Initial kernel scaffold; baseline (speedup 1.0000x reference)
import jax
import jax.numpy as jnp
from jax.experimental import pallas as pl


def kernel(x, weight, bias):
    raise NotImplementedError("write your pallas kernel here")


# speedup vs baseline: 1.3512x; 1.3512x over previous
import functools

import jax
import jax.numpy as jnp
from jax.experimental import pallas as pl
from jax.experimental.pallas import tpu as pltpu

B, C, H, W, O = 32, 64, 64, 64, 128
KH = KW = 3
HP, WP = H - KH + 1, W - KW + 1
ROWS = HP * W
XROWS = 4104
CK = C * KH * KW
NCORES = 2
BPC = B // NCORES


def _hebb_kernel(x_ref, wt_ref, b_ref, y_ref, a_ref, s_ref, p_scr):
    j = pl.program_id(1)

    for t in range(KH * KW):
        kh, kw = divmod(t, KW)
        off = kh * W + kw
        p_scr[:, t * C:(t + 1) * C] = x_ref[off:off + ROWS, :]

    wt = wt_ref[...]
    nrm = jnp.sqrt(jnp.sum(wt * wt, axis=0, keepdims=True))
    nrm = jnp.where(nrm == 0.0, 1.0, nrm)

    y = jnp.dot(p_scr[...], wt, preferred_element_type=jnp.float32)
    y = y / nrm + b_ref[...]
    y_ref[...] = y

    m = jnp.max(y, axis=1, keepdims=True)
    e = jnp.exp(y - m)
    r = e / jnp.sum(e, axis=1, keepdims=True)

    rowid = jax.lax.broadcasted_iota(jnp.int32, (ROWS, 1), 0)
    rm = jnp.where(rowid % W < WP, r, 0.0)
    r2 = rm * rm

    s1 = jnp.sum(rm, axis=0, keepdims=True)
    s2 = jnp.sum(r2, axis=0, keepdims=True)
    a = jax.lax.dot_general(r2, p_scr[...], (((0,), (0,)), ((), ())),
                            preferred_element_type=jnp.float32)

    @pl.when(j == 0)
    def _init():
        a_ref[...] = a
        s_ref[0:1, :] = s1
        s_ref[1:2, :] = s2

    @pl.when(j > 0)
    def _acc():
        a_ref[...] += a
        s_ref[0:1, :] += s1
        s_ref[1:2, :] += s2


@jax.jit
def kernel(x, weight, bias):
    xr = x.transpose(0, 2, 3, 1).reshape(B, H * W, C)
    xr = jnp.pad(xr, ((0, 0), (0, XROWS - H * W), (0, 0)))
    wt = weight.transpose(2, 3, 1, 0).reshape(CK, O)
    b2 = bias.reshape(1, O)

    y_flat, a_part, s_part = pl.pallas_call(
        _hebb_kernel,
        grid=(NCORES, BPC),
        in_specs=[
            pl.BlockSpec((None, XROWS, C), lambda i, j: (i * BPC + j, 0, 0)),
            pl.BlockSpec((CK, O), lambda i, j: (0, 0)),
            pl.BlockSpec((1, O), lambda i, j: (0, 0)),
        ],
        out_specs=[
            pl.BlockSpec((None, ROWS, O), lambda i, j: (i * BPC + j, 0, 0)),
            pl.BlockSpec((None, O, CK), lambda i, j: (i, 0, 0)),
            pl.BlockSpec((None, 8, O), lambda i, j: (i, 0, 0)),
        ],
        out_shape=[
            jax.ShapeDtypeStruct((B, ROWS, O), jnp.float32),
            jax.ShapeDtypeStruct((NCORES, O, CK), jnp.float32),
            jax.ShapeDtypeStruct((NCORES, 8, O), jnp.float32),
        ],
        scratch_shapes=[pltpu.VMEM((ROWS, CK), jnp.float32)],
        compiler_params=pltpu.CompilerParams(
            dimension_semantics=("parallel", "arbitrary"),
            vmem_limit_bytes=56 * 1024 * 1024,
        ),
    )(xr, wt, b2)

    y = y_flat.reshape(B, HP, W, O)[:, :, :WP, :].transpose(0, 3, 1, 2)

    a = a_part.sum(axis=0)
    s1 = s_part[:, 0, :].sum(axis=0)
    s2 = s_part[:, 1, :].sum(axis=0)
    r_sum = jnp.where(s1 == 0.0, 1.0, s1)
    a = a.reshape(O, KH, KW, C).transpose(0, 3, 1, 2)
    scale = (1.0 / r_sum)[:, None, None, None]
    delta_w = a * scale - (s2[:, None, None, None] * scale) * weight
    return y, delta_w


# speedup vs baseline: 1.7785x; 1.3162x over previous
import jax
import jax.numpy as jnp
from jax.experimental import pallas as pl
from jax.experimental.pallas import tpu as pltpu

B, C, H, W, O = 32, 64, 64, 64, 128
KH = KW = 3
HP, WP = H - KH + 1, W - KW + 1
COLS = HP * W
HW = H * W
CKE = C * KH * KW + 8
NCORES = 2
BPC = B // NCORES


def _hebb_kernel(x_ref, wq_ref, y_ref, a_ref, s_ref, pt_scr):
    j = pl.program_id(1)

    @pl.when(j == 0)
    def _tail():
        rid = jax.lax.broadcasted_iota(jnp.int32, (8, COLS), 0)
        pt_scr[C * KH * KW:CKE, :] = jnp.where(rid == 0, 1.0, 0.0)

    for t in range(KH * KW):
        kh, kw = divmod(t, KW)
        off = kh * W + kw
        n = min(HW - off, COLS)
        pt_scr[t * C:(t + 1) * C, 0:n] = x_ref[:, off:off + n]
        if n < COLS:
            pt_scr[t * C:(t + 1) * C, n:COLS] = jnp.zeros((C, COLS - n),
                                                          jnp.float32)

    wq = wq_ref[...]
    lid = jax.lax.broadcasted_iota(jnp.int32, (1, CKE), 1)
    wsq = jnp.where(lid < C * KH * KW, wq * wq, 0.0)
    nrm = jnp.sqrt(jnp.sum(wsq, axis=1, keepdims=True))
    rn = jnp.where(nrm == 0.0, 1.0, 1.0 / nrm)
    wn = jnp.where(lid < C * KH * KW, wq * rn, wq)

    y = jnp.dot(wn, pt_scr[...], preferred_element_type=jnp.float32)

    m = jnp.max(y, axis=0, keepdims=True)
    e = jnp.exp(y - m)
    r = e / jnp.sum(e, axis=0, keepdims=True)

    cid = jax.lax.broadcasted_iota(jnp.int32, (1, COLS), 1)
    rm = jnp.where(cid % W < WP, r, 0.0)
    r2 = rm * rm

    s1 = jnp.sum(rm, axis=1, keepdims=True)
    s2 = jnp.sum(r2, axis=1, keepdims=True)
    a = jax.lax.dot_general(r2, pt_scr[...], (((1,), (1,)), ((), ())),
                            preferred_element_type=jnp.float32)

    for i in range(HP):
        y_ref[:, i * WP:(i + 1) * WP] = y[:, i * W:i * W + WP]

    sj = jnp.concatenate([s1, s2], axis=1)

    @pl.when(j == 0)
    def _init():
        a_ref[...] = a
        s_ref[...] = sj

    @pl.when(j > 0)
    def _acc():
        a_ref[...] += a
        s_ref[...] += sj


@jax.jit
def kernel(x, weight, bias):
    xv = x.reshape(B, C, HW)
    wflat = weight.transpose(0, 2, 3, 1).reshape(O, C * KH * KW)
    wq = jnp.concatenate(
        [wflat, bias.reshape(O, 1), jnp.zeros((O, 7), jnp.float32)], axis=1)

    y_flat, a_part, s_part = pl.pallas_call(
        _hebb_kernel,
        grid=(NCORES, BPC),
        in_specs=[
            pl.BlockSpec((None, C, HW), lambda i, j: (i * BPC + j, 0, 0)),
            pl.BlockSpec((O, CKE), lambda i, j: (0, 0)),
        ],
        out_specs=[
            pl.BlockSpec((None, O, HP * WP), lambda i, j: (i * BPC + j, 0, 0)),
            pl.BlockSpec((None, O, CKE), lambda i, j: (i, 0, 0)),
            pl.BlockSpec((None, O, 2), lambda i, j: (i, 0, 0)),
        ],
        out_shape=[
            jax.ShapeDtypeStruct((B, O, HP * WP), jnp.float32),
            jax.ShapeDtypeStruct((NCORES, O, CKE), jnp.float32),
            jax.ShapeDtypeStruct((NCORES, O, 2), jnp.float32),
        ],
        scratch_shapes=[pltpu.VMEM((CKE, COLS), jnp.float32)],
        compiler_params=pltpu.CompilerParams(
            dimension_semantics=("parallel", "arbitrary"),
            vmem_limit_bytes=56 * 1024 * 1024,
        ),
    )(xv, wq)

    y = y_flat.reshape(B, O, HP, WP)

    a = a_part.sum(axis=0)[:, :C * KH * KW]
    s1 = s_part[:, :, 0].sum(axis=0)
    s2 = s_part[:, :, 1].sum(axis=0)
    r_sum = jnp.where(s1 == 0.0, 1.0, s1)
    a = a.reshape(O, KH, KW, C).transpose(0, 3, 1, 2)
    scale = (1.0 / r_sum)[:, None, None, None]
    delta_w = a * scale - (s2[:, None, None, None] * scale) * weight
    return y, delta_w
